# baseline (speedup 1.0000x reference)
import functools

import jax
import jax.numpy as jnp
from jax import lax
from jax.experimental import pallas as pl
from jax.experimental.pallas import tpu as pltpu
from jax.experimental.pallas import tpu_sc as plsc

_VOCAB = 1000000
_DIM = 64
_BATCH = 4096
_HIST = 200

_NC = 2
_NS = 16
_NW = _NC * _NS

_B_TOTAL = _BATCH * _HIST
_B_PER_W = _B_TOTAL // _NW
_CHUNK = 128
_N_CHUNKS = _B_PER_W // _CHUNK


@functools.partial(
    pl.kernel,
    out_type=jax.ShapeDtypeStruct((_B_TOTAL, _DIM), jnp.float32),
    mesh=plsc.VectorSubcoreMesh(
        core_axis_name="c", subcore_axis_name="s", num_cores=_NC, num_subcores=_NS
    ),
    compiler_params=pltpu.CompilerParams(use_tc_tiling_on_sc=False),
    scratch_types=[
        pltpu.VMEM((_N_CHUNKS, _CHUNK), jnp.int32),
        pltpu.VMEM((2, _CHUNK, _DIM), jnp.float32),
        pltpu.SemaphoreType.DMA,
        pltpu.SemaphoreType.DMA,
    ],
)
def _gather_kernel(table_hbm, tok_hbm, out_hbm, idx_v, rows_v, gsem, wsem):
    wid = lax.axis_index("s") * _NC + lax.axis_index("c")
    base = wid * _B_PER_W

    pltpu.sync_copy(tok_hbm.at[wid], idx_v)

    pltpu.async_copy(table_hbm.at[idx_v.at[0]], rows_v.at[0], gsem)

    def chunk_body(i, _):
        slot = lax.rem(i, 2)
        nxt = lax.rem(i + 1, 2)

        @pl.when(i >= 1)
        def _():
            pltpu.make_async_copy(
                rows_v.at[nxt],
                out_hbm.at[pl.ds(base + (i - 1) * _CHUNK, _CHUNK)],
                wsem,
            ).wait()

        @pl.when(i + 1 < _N_CHUNKS)
        def _():
            pltpu.async_copy(table_hbm.at[idx_v.at[i + 1]], rows_v.at[nxt], gsem)

        pltpu.make_async_copy(table_hbm.at[idx_v.at[i]], rows_v.at[slot], gsem).wait()

        pltpu.async_copy(
            rows_v.at[slot], out_hbm.at[pl.ds(base + i * _CHUNK, _CHUNK)], wsem
        )
        return 0

    lax.fori_loop(0, _N_CHUNKS, chunk_body, 0)

    pltpu.make_async_copy(
        rows_v.at[(_N_CHUNKS - 1) % 2],
        out_hbm.at[pl.ds(base + (_N_CHUNKS - 1) * _CHUNK, _CHUNK)],
        wsem,
    ).wait()


def kernel(tokens, embedding_weight):
    tok = tokens.astype(jnp.int32).reshape(_NW, _N_CHUNKS, _CHUNK)
    out = _gather_kernel(embedding_weight, tok)
    return out.reshape(_BATCH, _HIST, _DIM)
